# Initial kernel scaffold; baseline (speedup 1.0000x reference)
import jax
import jax.numpy as jnp
from jax.experimental import pallas as pl


def kernel(x, coords, edge_index, data_edge_index, Wa, ba, Wg, bg, Waf, baf, Wc1, bc1, Wm1, bm1, Wc2, bc2, Wf1, bf1, Wf2, bf2):
    raise NotImplementedError("write your pallas kernel here")


# speedup vs baseline: 9.3767x; 9.3767x over previous
import functools
import jax
import jax.numpy as jnp
from jax import lax
from jax.experimental import pallas as pl
from jax.experimental.pallas import tpu as pltpu
from jax.experimental.pallas import tpu_sc as plsc

N = 10000
E = 150000
NPAD = 10240
EPAD = 153600
EPT = EPAD // 16
EPW = EPAD // 32
SPH = 3200
ROWS_PT = NPAD // 16
RB = 2048
L = 16

_SC_PARAMS = pltpu.CompilerParams(needs_layout_passes=False)


def _f32(shape):
    return jax.ShapeDtypeStruct(shape, jnp.float32)


def _mesh():
    return plsc.VectorSubcoreMesh(core_axis_name="c", subcore_axis_name="s")


def _tc1_body(x_ref, c_ref, wc1t_ref, wn_ref, wg_ref, hwh_ref, ns_ref):
    xb = x_ref[...]
    hw = jnp.dot(xb, wc1t_ref[...], preferred_element_type=jnp.float32)
    hwh_ref[0] = hw[:, :128]
    hwh_ref[1] = hw[:, 128:]
    ns_ref[...] = jnp.concatenate(
        [jnp.dot(xb, wn_ref[...], preferred_element_type=jnp.float32),
         jnp.dot(c_ref[...], wg_ref[...], preferred_element_type=jnp.float32)],
        axis=1)


def _tc2_body(h0_ref, h1_ref, w2a_ref, w2b_ref, wm_ref, hw_ref, ms_ref):
    h0 = h0_ref[0]
    h1 = h1_ref[0]
    hw_ref[...] = (jnp.dot(h0, w2a_ref[...], preferred_element_type=jnp.float32)
                   + jnp.dot(h1, w2b_ref[...], preferred_element_type=jnp.float32))
    ms_ref[...] = (jnp.dot(h0, wm_ref[...][:128], preferred_element_type=jnp.float32)
                   + jnp.dot(h1, wm_ref[...][128:], preferred_element_type=jnp.float32))


def _tc3_body(o0_ref, o1_ref, bc2_ref, f1t_ref, p_ref):
    out = o0_ref[0] + o1_ref[0] + bc2_ref[...]
    p = jnp.dot(out, f1t_ref[...], preferred_element_type=jnp.float32)
    p_ref[...] = jnp.concatenate([p, jnp.zeros_like(p)], axis=1)


def _tc4_body(q_ref, bf1_ref, wf2_ref, bf2_ref, out_ref):
    hid = jnp.maximum(q_ref[...] + bf1_ref[...], 0.0)
    logit = jnp.dot(hid, wf2_ref[...], preferred_element_type=jnp.float32) + bf2_ref[0, 0]
    out_ref[...] = jax.nn.sigmoid(logit)


def _rsqrt16(d):
    i = plsc.bitcast(d, jnp.int32)
    i = jnp.int32(0x5F3759DF) - (i >> 1)
    y = plsc.bitcast(i, jnp.float32)
    for _ in range(3):
        y = y * (1.5 - 0.5 * d * y * y)
    return y


def _bcast(ref, r):
    return plsc.load_gather(ref, [jnp.full((L,), r, jnp.int32)])


def _make_edge(two_layer):
    tc = 4 if two_layer else 2
    nch = EPW // 128
    tail = EPW - nch * 128

    @functools.partial(
        pl.kernel,
        out_type=(_f32((EPAD,)), _f32((2, NPAD))),
        mesh=_mesh(),
        compiler_params=_SC_PARAMS,
        scratch_types=dict(
            tbl_v=pltpu.VMEM((NPAD * tc,), jnp.float32),
            src_v=pltpu.VMEM((EPW,), jnp.int32),
            dst_v=pltpu.VMEM((EPW,), jnp.int32),
            ew_v=pltpu.VMEM((EPW,), jnp.float32),
            dstbuf=pltpu.VMEM((1, 128), jnp.int32),
            dstbuf_t=pltpu.VMEM((1, tail), jnp.int32),
            slice_v=pltpu.VMEM((ROWS_PT,), jnp.float32),
            consts_v=pltpu.VMEM((8, L), jnp.float32),
            deg_sp=pltpu.VMEM_SHARED((NPAD,), jnp.float32),
        ),
    )
    def edge(tbl_h, src_h, dst_h, consts_h, ew_h, degpart_h,
             tbl_v, src_v, dst_v, ew_v, dstbuf, dstbuf_t, slice_v,
             consts_v, deg_sp):
        cid = lax.axis_index("c")
        sid = lax.axis_index("s")
        ebase = (sid * 2 + cid) * EPW
        rbase = sid * ROWS_PT

        pltpu.sync_copy(tbl_h, tbl_v)
        pltpu.sync_copy(src_h.at[pl.ds(ebase, EPW)], src_v)
        pltpu.sync_copy(dst_h.at[pl.ds(ebase, EPW)], dst_v)
        pltpu.sync_copy(consts_h, consts_v)

        init = jnp.where(cid == 0, 1.0, 0.0)

        def _fill(i, _):
            slice_v[pl.ds(i * L, L)] = jnp.full((L,), init, jnp.float32)
            return _
        lax.fori_loop(0, ROWS_PT // L, _fill, None)
        pltpu.sync_copy(slice_v, deg_sp.at[pl.ds(rbase, ROWS_PT)])
        plsc.subcore_barrier()

        c0 = consts_v[0]
        c1 = consts_v[1]
        c2 = consts_v[2]
        c3 = consts_v[3]
        c4 = consts_v[4]

        def _ew16(o):
            d16r = dst_v[pl.ds(o, L)]
            s16 = src_v[pl.ds(o, L)] * tc
            d16 = d16r * tc
            if two_layer:
                sA = plsc.load_gather(tbl_v, [s16])
                dA = plsc.load_gather(tbl_v, [d16 + 1])
                sG = plsc.load_gather(tbl_v, [s16 + 2])
                dG = plsc.load_gather(tbl_v, [d16 + 3])
                x1 = jnp.maximum(sA + dA + c0, 0.0)
                x2 = jnp.maximum(sG + dG + c1, 0.0)
                ew = jnp.maximum(c3 * x1 + c4 * x2 + c2, 0.0)
            else:
                sA = plsc.load_gather(tbl_v, [s16])
                dA = plsc.load_gather(tbl_v, [d16 + 1])
                ew = jnp.maximum(sA + dA + c0, 0.0)
            gid = ebase + o + lax.iota(jnp.int32, L)
            ew = jnp.where(gid < E, ew, 0.0)
            ew_v[pl.ds(o, L)] = ew
            return d16r

        def _chunk(k, _):
            eb = k * 128
            for j in range(8):
                dstbuf[0, pl.ds(j * L, L)] = _ew16(eb + j * L)
            pltpu.sync_copy(ew_v.at[pl.ds(eb, 128)],
                            deg_sp.at[dstbuf.at[0]], add=True)
            return _
        lax.fori_loop(0, nch, _chunk, None)
        for j in range(tail // L):
            dstbuf_t[0, pl.ds(j * L, L)] = _ew16(nch * 128 + j * L)
        pltpu.sync_copy(ew_v.at[pl.ds(nch * 128, tail)],
                        deg_sp.at[dstbuf_t.at[0]], add=True)

        pltpu.sync_copy(ew_v, ew_h.at[pl.ds(ebase, EPW)])
        plsc.subcore_barrier()

        pltpu.sync_copy(deg_sp.at[pl.ds(rbase, ROWS_PT)], slice_v)
        pltpu.sync_copy(slice_v, degpart_h.at[cid].at[pl.ds(rbase, ROWS_PT)])

    return edge


def _make_agg(width, relu_out):
    blocks = ROWS_PT // 128
    nsp = EPT // SPH
    nch = SPH // 128
    npc = 2560

    @functools.partial(
        pl.kernel,
        out_type=_f32((2, NPAD, width)),
        mesh=_mesh(),
        compiler_params=_SC_PARAMS,
        scratch_types=dict(
            src_v=pltpu.VMEM((SPH,), jnp.int32),
            dst_v=pltpu.VMEM((SPH,), jnp.int32),
            ew_v=pltpu.VMEM((SPH,), jnp.float32),
            dis_v=pltpu.VMEM((NPAD,), jnp.float32),
            tmp_v=pltpu.VMEM((npc,), jnp.float32),
            rowbuf=pltpu.VMEM((128, width), jnp.float32),
            normbuf=pltpu.VMEM((128,), jnp.float32),
            dstbuf=pltpu.VMEM((1, 128), jnp.int32),
            bc_v=pltpu.VMEM((width,), jnp.float32),
            acc_sp=pltpu.VMEM_SHARED((NPAD, width), jnp.float32),
            sem=pltpu.SemaphoreType.DMA,
        ),
    )
    def agg(src_h, dst_h, ew_h, degpart_h, hwh_h, bch_h, out_h,
            src_v, dst_v, ew_v, dis_v, tmp_v, rowbuf, normbuf, dstbuf,
            bc_v, acc_sp, sem):
        cid = lax.axis_index("c")
        sid = lax.axis_index("s")
        ebase = sid * EPT
        rbase = sid * ROWS_PT

        pltpu.sync_copy(degpart_h.at[0], dis_v)
        pltpu.sync_copy(bch_h.at[cid], bc_v)

        for piece in range(NPAD // npc):
            pltpu.sync_copy(degpart_h.at[1].at[pl.ds(piece * npc, npc)], tmp_v)

            def _sum_chunk(i, _):
                o = piece * npc + i * L
                d = dis_v[pl.ds(o, L)] + tmp_v[pl.ds(i * L, L)]
                dis_v[pl.ds(o, L)] = _rsqrt16(d)
                return _
            lax.fori_loop(0, npc // L, _sum_chunk, None)

        for b in range(blocks):
            rows = rbase + b * 128
            pltpu.sync_copy(hwh_h.at[cid].at[pl.ds(rows, 128)], rowbuf)

            def _init_row(r, _):
                g = _bcast(dis_v, rows + r)
                rec = g * g
                for j in range(width // L):
                    rowbuf[r, pl.ds(j * L, L)] = rowbuf[r, pl.ds(j * L, L)] * rec
                return _
            lax.fori_loop(0, 128, _init_row, None)
            pltpu.sync_copy(rowbuf, acc_sp.at[pl.ds(rows, 128)])
        plsc.subcore_barrier()

        for sp in range(nsp):
            sbase = ebase + sp * SPH
            pltpu.sync_copy(src_h.at[pl.ds(sbase, SPH)], src_v)
            pltpu.sync_copy(dst_h.at[pl.ds(sbase, SPH)], dst_v)
            pltpu.sync_copy(ew_h.at[pl.ds(sbase, SPH)], ew_v)

            def _agg_chunk(k, _):
                eb = k * 128
                pltpu.async_copy(
                    hwh_h.at[cid].at[src_v.at[pl.ds(eb, 128)]], rowbuf,
                    sem).wait()
                for j in range(8):
                    o = eb + j * L
                    s16 = src_v[pl.ds(o, L)]
                    d16 = dst_v[pl.ds(o, L)]
                    n16 = (plsc.load_gather(dis_v, [s16]) * ew_v[pl.ds(o, L)]
                           * plsc.load_gather(dis_v, [d16]))
                    normbuf[pl.ds(j * L, L)] = n16
                    dstbuf[0, pl.ds(j * L, L)] = d16

                def _scale_row(r, _):
                    nb = _bcast(normbuf, r)
                    for j in range(width // L):
                        rowbuf[r, pl.ds(j * L, L)] = (
                            rowbuf[r, pl.ds(j * L, L)] * nb)
                    return _
                lax.fori_loop(0, 128, _scale_row, None)
                pltpu.sync_copy(rowbuf, acc_sp.at[dstbuf.at[0]], add=True)
                return _
            lax.fori_loop(0, nch, _agg_chunk, None)
        plsc.subcore_barrier()

        for b in range(blocks):
            rows = rbase + b * 128
            pltpu.sync_copy(acc_sp.at[pl.ds(rows, 128)], rowbuf)

            def _out_row(r, _):
                for j in range(width // L):
                    v = rowbuf[r, pl.ds(j * L, L)] + bc_v[pl.ds(j * L, L)]
                    if relu_out:
                        v = jnp.maximum(v, 0.0)
                    rowbuf[r, pl.ds(j * L, L)] = v
                return _
            lax.fori_loop(0, 128, _out_row, None)
            pltpu.sync_copy(rowbuf, out_h.at[cid].at[pl.ds(rows, 128)])

    return agg


def _make_agg2():
    width = 128
    blocks = ROWS_PT // 128
    nch = EPW // 64
    npc = 2560

    @functools.partial(
        pl.kernel,
        out_type=_f32((2, NPAD, width)),
        mesh=_mesh(),
        compiler_params=_SC_PARAMS,
        scratch_types=dict(
            src_v=pltpu.VMEM((EPW,), jnp.int32),
            dst_v=pltpu.VMEM((EPW,), jnp.int32),
            ew_v=pltpu.VMEM((EPW,), jnp.float32),
            dis_v=pltpu.VMEM((NPAD,), jnp.float32),
            tmp_v=pltpu.VMEM((npc,), jnp.float32),
            rowbuf=pltpu.VMEM((64, width), jnp.float32),
            normbuf=pltpu.VMEM((64,), jnp.float32),
            dstbuf=pltpu.VMEM((1, 64), jnp.int32),
            acc_sp=pltpu.VMEM_SHARED((NPAD, width), jnp.float32),
            sem=pltpu.SemaphoreType.DMA,
        ),
    )
    def agg2(src_h, dst_h, ew_h, degpart_h, hw_h, part_h,
             src_v, dst_v, ew_v, dis_v, tmp_v, rowbuf, normbuf, dstbuf,
             acc_sp, sem):
        cid = lax.axis_index("c")
        sid = lax.axis_index("s")
        ebase = (sid * 2 + cid) * EPW
        rbase = sid * ROWS_PT

        pltpu.sync_copy(src_h.at[pl.ds(ebase, EPW)], src_v)
        pltpu.sync_copy(dst_h.at[pl.ds(ebase, EPW)], dst_v)
        pltpu.sync_copy(ew_h.at[pl.ds(ebase, EPW)], ew_v)
        pltpu.sync_copy(degpart_h.at[0], dis_v)

        for piece in range(NPAD // npc):
            pltpu.sync_copy(degpart_h.at[1].at[pl.ds(piece * npc, npc)], tmp_v)

            def _sum_chunk(i, _):
                o = piece * npc + i * L
                d = dis_v[pl.ds(o, L)] + tmp_v[pl.ds(i * L, L)]
                dis_v[pl.ds(o, L)] = _rsqrt16(d)
                return _
            lax.fori_loop(0, npc // L, _sum_chunk, None)

        cz = jnp.where(cid == 0, 1.0, 0.0)
        for b in range(ROWS_PT // 64):
            rows = rbase + b * 64
            pltpu.sync_copy(hw_h.at[pl.ds(rows, 64)], rowbuf)

            def _init_row(r, _):
                g = _bcast(dis_v, rows + r)
                rec = g * g * cz
                for j in range(width // L):
                    rowbuf[r, pl.ds(j * L, L)] = rowbuf[r, pl.ds(j * L, L)] * rec
                return _
            lax.fori_loop(0, 64, _init_row, None)
            pltpu.sync_copy(rowbuf, acc_sp.at[pl.ds(rows, 64)])
        plsc.subcore_barrier()

        def _agg_chunk(k, _):
            eb = k * 64
            pltpu.async_copy(
                hw_h.at[src_v.at[pl.ds(eb, 64)]], rowbuf, sem).wait()
            for j in range(4):
                o = eb + j * L
                s16 = src_v[pl.ds(o, L)]
                d16 = dst_v[pl.ds(o, L)]
                n16 = (plsc.load_gather(dis_v, [s16]) * ew_v[pl.ds(o, L)]
                       * plsc.load_gather(dis_v, [d16]))
                normbuf[pl.ds(j * L, L)] = n16
                dstbuf[0, pl.ds(j * L, L)] = d16

            def _scale_row(r, _):
                nb = _bcast(normbuf, r)
                for j in range(width // L):
                    rowbuf[r, pl.ds(j * L, L)] = rowbuf[r, pl.ds(j * L, L)] * nb
                return _
            lax.fori_loop(0, 64, _scale_row, None)
            pltpu.sync_copy(rowbuf, acc_sp.at[dstbuf.at[0]], add=True)
            return _
        lax.fori_loop(0, nch, _agg_chunk, None)
        plsc.subcore_barrier()

        for b in range(ROWS_PT // 64):
            rows = rbase + b * 64
            pltpu.sync_copy(acc_sp.at[pl.ds(rows, 64)], rowbuf)
            pltpu.sync_copy(rowbuf, part_h.at[cid].at[pl.ds(rows, 64)])

    return agg2


def _make_final():
    nch = EPW // 64

    @functools.partial(
        pl.kernel,
        out_type=_f32((EPAD, 64)),
        mesh=_mesh(),
        compiler_params=_SC_PARAMS,
        scratch_types=dict(
            d0_v=pltpu.VMEM((EPW,), jnp.int32),
            d1_v=pltpu.VMEM((EPW,), jnp.int32),
            buf0=pltpu.VMEM((64, 128), jnp.float32),
            buf1=pltpu.VMEM((64, 128), jnp.float32),
            qbuf=pltpu.VMEM((64, 64), jnp.float32),
            p_sp=pltpu.VMEM_SHARED((NPAD, 128), jnp.float32),
            sem=pltpu.SemaphoreType.DMA,
        ),
    )
    def final(p_h, d0_h, d1_h, q_h, d0_v, d1_v, buf0, buf1, qbuf, p_sp, sem):
        cid = lax.axis_index("c")
        sid = lax.axis_index("s")
        base = (sid * 2 + cid) * EPW

        for b in range(ROWS_PT // 64):
            rows = sid * ROWS_PT + b * 64
            pltpu.sync_copy(p_h.at[pl.ds(rows, 64)], buf0)
            pltpu.sync_copy(buf0, p_sp.at[pl.ds(rows, 64)])
        pltpu.sync_copy(d0_h.at[pl.ds(base, EPW)], d0_v)
        pltpu.sync_copy(d1_h.at[pl.ds(base, EPW)], d1_v)
        plsc.subcore_barrier()

        def _chunk(k, _):
            eb = k * 64
            pltpu.async_copy(p_sp.at[d0_v.at[pl.ds(eb, 64)]], buf0, sem).wait()
            pltpu.async_copy(p_sp.at[d1_v.at[pl.ds(eb, 64)]], buf1, sem).wait()

            def _sub_row(r, _):
                for j in range(4):
                    qbuf[r, pl.ds(j * L, L)] = (buf0[r, pl.ds(j * L, L)]
                                                - buf1[r, pl.ds(j * L, L)])
                return _
            lax.fori_loop(0, 64, _sub_row, None)
            pltpu.sync_copy(qbuf, q_h.at[pl.ds(base + eb, 64)])
            return _
        lax.fori_loop(0, nch, _chunk, None)

    return final


_edge1 = _make_edge(True)
_edge2 = _make_edge(False)
_agg1 = _make_agg(128, True)
_agg2 = _make_agg2()
_final = _make_final()


def kernel(x, coords, edge_index, data_edge_index, Wa, ba, Wg, bg, Waf, baf,
           Wc1, bc1, Wm1, bm1, Wc2, bc2, Wf1, bf1, Wf2, bf2):
    f32 = jnp.float32
    xp = jnp.zeros((NPAD, 512), f32).at[:N].set(x)
    cp = jnp.zeros((NPAD, 4), f32).at[:N].set(coords)
    pad_idx = (N + jnp.arange(EPAD - E, dtype=jnp.int32) % (NPAD - N))
    srcp = jnp.concatenate([edge_index[0], pad_idx])
    dstp = jnp.concatenate([edge_index[1], pad_idx])
    d0p = jnp.concatenate([data_edge_index[0], pad_idx])
    d1p = jnp.concatenate([data_edge_index[1], pad_idx])

    grid = NPAD // RB
    wnode = Wa[0].reshape(2, 512).T
    wgnode = Wg[0].reshape(2, 4).T
    hwh1, ns = pl.pallas_call(
        _tc1_body,
        grid=(grid,),
        in_specs=[
            pl.BlockSpec((RB, 512), lambda i: (i, 0)),
            pl.BlockSpec((RB, 4), lambda i: (i, 0)),
            pl.BlockSpec((512, 256), lambda i: (0, 0)),
            pl.BlockSpec((512, 2), lambda i: (0, 0)),
            pl.BlockSpec((4, 2), lambda i: (0, 0)),
        ],
        out_specs=[
            pl.BlockSpec((2, RB, 128), lambda i: (0, i, 0)),
            pl.BlockSpec((RB, 4), lambda i: (i, 0)),
        ],
        out_shape=[_f32((2, NPAD, 128)), _f32((NPAD, 4))],
    )(xp, cp, Wc1.T, wnode, wgnode)

    z16 = jnp.zeros((L,), f32)
    consts1 = jnp.stack([
        z16 + ba[0], z16 + bg[0], z16 + baf[0],
        z16 + Waf[0, 0], z16 + Waf[0, 1],
        z16, z16, z16])
    ew1, degpart1 = _edge1(ns.reshape(-1), srcp, dstp, consts1)
    h = _agg1(srcp, dstp, ew1, degpart1, hwh1, bc1.reshape(2, 128))

    wm = Wm1[0].reshape(2, 256).T
    hwh2, ms = pl.pallas_call(
        _tc2_body,
        grid=(grid,),
        in_specs=[
            pl.BlockSpec((1, RB, 128), lambda i: (0, i, 0)),
            pl.BlockSpec((1, RB, 128), lambda i: (1, i, 0)),
            pl.BlockSpec((128, 128), lambda i: (0, 0)),
            pl.BlockSpec((128, 128), lambda i: (0, 0)),
            pl.BlockSpec((256, 2), lambda i: (0, 0)),
        ],
        out_specs=[
            pl.BlockSpec((RB, 128), lambda i: (i, 0)),
            pl.BlockSpec((RB, 2), lambda i: (i, 0)),
        ],
        out_shape=[_f32((NPAD, 128)), _f32((NPAD, 2))],
    )(h, h, Wc2[:, :128].T, Wc2[:, 128:].T, wm)

    consts2 = jnp.stack([z16 + bm1[0]] + [z16] * 7)
    ew2, degpart2 = _edge2(ms.reshape(-1), srcp, dstp, consts2)
    part = _agg2(srcp, dstp, ew2, degpart2, hwh2)

    p = pl.pallas_call(
        _tc3_body,
        grid=(grid,),
        in_specs=[
            pl.BlockSpec((1, RB, 128), lambda i: (0, i, 0)),
            pl.BlockSpec((1, RB, 128), lambda i: (1, i, 0)),
            pl.BlockSpec((1, 128), lambda i: (0, 0)),
            pl.BlockSpec((128, 64), lambda i: (0, 0)),
        ],
        out_specs=pl.BlockSpec((RB, 128), lambda i: (i, 0)),
        out_shape=_f32((NPAD, 128)),
    )(part, part, bc2.reshape(1, 128), Wf1.T)

    q = _final(p, d0p, d1p)

    prob = pl.pallas_call(
        _tc4_body,
        grid=(EPAD // RB,),
        in_specs=[
            pl.BlockSpec((RB, 64), lambda i: (i, 0)),
            pl.BlockSpec((1, 64), lambda i: (0, 0)),
            pl.BlockSpec((64, 1), lambda i: (0, 0)),
            pl.BlockSpec((1, 1), lambda i: (0, 0)),
        ],
        out_specs=pl.BlockSpec((RB, 1), lambda i: (i, 0)),
        out_shape=_f32((EPAD, 1)),
    )(q, bf1.reshape(1, 64), Wf2.reshape(64, 1), bf2.reshape(1, 1))

    return prob[:E]


# speedup vs baseline: 10.7672x; 1.1483x over previous
import functools
import jax
import jax.numpy as jnp
from jax import lax
from jax.experimental import pallas as pl
from jax.experimental.pallas import tpu as pltpu
from jax.experimental.pallas import tpu_sc as plsc

N = 10000
E = 150000
NPAD = 10240
EPAD = 153600
EPT = EPAD // 16
EPW = EPAD // 32
ROWS_PT = NPAD // 16
RB = 2048
L = 16
CH = 64

_SC_PARAMS = pltpu.CompilerParams(needs_layout_passes=False)


def _f32(shape):
    return jax.ShapeDtypeStruct(shape, jnp.float32)


def _mesh():
    return plsc.VectorSubcoreMesh(core_axis_name="c", subcore_axis_name="s")


def _tc1_body(x_ref, c_ref, wc1t_ref, wn_ref, wg_ref, hwh_ref, ns_ref):
    xb = x_ref[...]
    hw = jnp.dot(xb, wc1t_ref[...], preferred_element_type=jnp.float32)
    hwh_ref[0] = hw[:, :128]
    hwh_ref[1] = hw[:, 128:]
    ns_ref[...] = jnp.concatenate(
        [jnp.dot(xb, wn_ref[...], preferred_element_type=jnp.float32),
         jnp.dot(c_ref[...], wg_ref[...], preferred_element_type=jnp.float32)],
        axis=1)


def _tc2_body(h0_ref, h1_ref, w2a_ref, w2b_ref, wm_ref, hw_ref, ms_ref):
    h0 = h0_ref[0]
    h1 = h1_ref[0]
    hw_ref[...] = (jnp.dot(h0, w2a_ref[...], preferred_element_type=jnp.float32)
                   + jnp.dot(h1, w2b_ref[...], preferred_element_type=jnp.float32))
    ms_ref[...] = (jnp.dot(h0, wm_ref[...][:128], preferred_element_type=jnp.float32)
                   + jnp.dot(h1, wm_ref[...][128:], preferred_element_type=jnp.float32))


def _tc3_body(o0_ref, o1_ref, bc2_ref, f1t_ref, p_ref):
    out = o0_ref[0] + o1_ref[0] + bc2_ref[...]
    p = jnp.dot(out, f1t_ref[...], preferred_element_type=jnp.float32)
    p_ref[...] = jnp.concatenate([p, jnp.zeros_like(p)], axis=1)


def _tc4_body(q_ref, bf1_ref, wf2_ref, bf2_ref, out_ref):
    hid = jnp.maximum(q_ref[...] + bf1_ref[...], 0.0)
    logit = jnp.dot(hid, wf2_ref[...], preferred_element_type=jnp.float32) + bf2_ref[0, 0]
    out_ref[...] = jax.nn.sigmoid(logit)


def _rsqrt16(d):
    i = plsc.bitcast(d, jnp.int32)
    i = jnp.int32(0x5F3759DF) - (i >> 1)
    y = plsc.bitcast(i, jnp.float32)
    for _ in range(3):
        y = y * (1.5 - 0.5 * d * y * y)
    return y


def _bcast(ref, r):
    return plsc.load_gather(ref, [jnp.full((L,), r, jnp.int32)])


def _make_edge(two_layer):
    tc = 4 if two_layer else 2
    nch = EPW // 128
    tail = EPW - nch * 128

    @functools.partial(
        pl.kernel,
        out_type=(_f32((EPAD,)), _f32((2, NPAD))),
        mesh=_mesh(),
        compiler_params=_SC_PARAMS,
        scratch_types=dict(
            tbl_v=pltpu.VMEM((NPAD * tc,), jnp.float32),
            src_v=pltpu.VMEM((EPW,), jnp.int32),
            dst_v=pltpu.VMEM((EPW,), jnp.int32),
            ew_v=pltpu.VMEM((EPW,), jnp.float32),
            dstbuf=pltpu.VMEM((1, 128), jnp.int32),
            dstbuf_t=pltpu.VMEM((1, tail), jnp.int32),
            slice_v=pltpu.VMEM((ROWS_PT,), jnp.float32),
            consts_v=pltpu.VMEM((8, L), jnp.float32),
            deg_sp=pltpu.VMEM_SHARED((NPAD,), jnp.float32),
        ),
    )
    def edge(tbl_h, src_h, dst_h, consts_h, ew_h, degpart_h,
             tbl_v, src_v, dst_v, ew_v, dstbuf, dstbuf_t, slice_v,
             consts_v, deg_sp):
        cid = lax.axis_index("c")
        sid = lax.axis_index("s")
        ebase = (sid * 2 + cid) * EPW
        rbase = sid * ROWS_PT

        pltpu.sync_copy(tbl_h, tbl_v)
        pltpu.sync_copy(src_h.at[pl.ds(ebase, EPW)], src_v)
        pltpu.sync_copy(dst_h.at[pl.ds(ebase, EPW)], dst_v)
        pltpu.sync_copy(consts_h, consts_v)

        init = jnp.where(cid == 0, 1.0, 0.0)

        def _fill(i, _):
            slice_v[pl.ds(i * L, L)] = jnp.full((L,), init, jnp.float32)
            return _
        lax.fori_loop(0, ROWS_PT // L, _fill, None)
        pltpu.sync_copy(slice_v, deg_sp.at[pl.ds(rbase, ROWS_PT)])
        plsc.subcore_barrier()

        c0 = consts_v[0]
        c1 = consts_v[1]
        c2 = consts_v[2]
        c3 = consts_v[3]
        c4 = consts_v[4]

        def _ew16(o):
            d16r = dst_v[pl.ds(o, L)]
            s16 = src_v[pl.ds(o, L)] * tc
            d16 = d16r * tc
            if two_layer:
                sA = plsc.load_gather(tbl_v, [s16])
                dA = plsc.load_gather(tbl_v, [d16 + 1])
                sG = plsc.load_gather(tbl_v, [s16 + 2])
                dG = plsc.load_gather(tbl_v, [d16 + 3])
                x1 = jnp.maximum(sA + dA + c0, 0.0)
                x2 = jnp.maximum(sG + dG + c1, 0.0)
                ew = jnp.maximum(c3 * x1 + c4 * x2 + c2, 0.0)
            else:
                sA = plsc.load_gather(tbl_v, [s16])
                dA = plsc.load_gather(tbl_v, [d16 + 1])
                ew = jnp.maximum(sA + dA + c0, 0.0)
            gid = ebase + o + lax.iota(jnp.int32, L)
            ew = jnp.where(gid < E, ew, 0.0)
            ew_v[pl.ds(o, L)] = ew
            return d16r

        def _chunk(k, _):
            eb = k * 128
            for j in range(8):
                dstbuf[0, pl.ds(j * L, L)] = _ew16(eb + j * L)
            pltpu.sync_copy(ew_v.at[pl.ds(eb, 128)],
                            deg_sp.at[dstbuf.at[0]], add=True)
            return _
        lax.fori_loop(0, nch, _chunk, None)
        for j in range(tail // L):
            dstbuf_t[0, pl.ds(j * L, L)] = _ew16(nch * 128 + j * L)
        pltpu.sync_copy(ew_v.at[pl.ds(nch * 128, tail)],
                        deg_sp.at[dstbuf_t.at[0]], add=True)

        pltpu.sync_copy(ew_v, ew_h.at[pl.ds(ebase, EPW)])
        plsc.subcore_barrier()

        pltpu.sync_copy(deg_sp.at[pl.ds(rbase, ROWS_PT)], slice_v)
        pltpu.sync_copy(slice_v, degpart_h.at[cid].at[pl.ds(rbase, ROWS_PT)])

    return edge


def _make_agg(feature_split):
    ept = EPT if feature_split else EPW
    sph = 1920 if feature_split else 960
    nsp = ept // sph
    nch = sph // CH
    ngr = nch // 3
    npc = 2560
    width = 128

    @functools.partial(
        pl.kernel,
        out_type=_f32((2, NPAD, width)),
        mesh=_mesh(),
        compiler_params=_SC_PARAMS,
        scratch_types=dict(
            src_v=pltpu.VMEM((sph,), jnp.int32),
            dst_v=pltpu.VMEM((sph,), jnp.int32),
            ew_v=pltpu.VMEM((sph,), jnp.float32),
            dis_v=pltpu.VMEM((NPAD,), jnp.float32),
            tmp_v=pltpu.VMEM((npc,), jnp.float32),
            buf0=pltpu.VMEM((CH, width), jnp.float32),
            buf1=pltpu.VMEM((CH, width), jnp.float32),
            buf2=pltpu.VMEM((CH, width), jnp.float32),
            db0=pltpu.VMEM((1, CH), jnp.int32),
            db1=pltpu.VMEM((1, CH), jnp.int32),
            db2=pltpu.VMEM((1, CH), jnp.int32),
            normbuf=pltpu.VMEM((CH,), jnp.float32),
            bc_v=pltpu.VMEM((width,), jnp.float32),
            acc_sp=pltpu.VMEM_SHARED((NPAD, width), jnp.float32),
            sg0=pltpu.SemaphoreType.DMA,
            sg1=pltpu.SemaphoreType.DMA,
            sg2=pltpu.SemaphoreType.DMA,
            ss0=pltpu.SemaphoreType.DMA,
            ss1=pltpu.SemaphoreType.DMA,
            ss2=pltpu.SemaphoreType.DMA,
        ),
    )
    def agg(src_h, dst_h, ew_h, degpart_h, hwh_h, bch_h, out_h,
            src_v, dst_v, ew_v, dis_v, tmp_v, buf0, buf1, buf2,
            db0, db1, db2, normbuf, bc_v, acc_sp, sg0, sg1, sg2,
            ss0, ss1, ss2):
        cid = lax.axis_index("c")
        sid = lax.axis_index("s")
        if feature_split:
            ebase = sid * ept
            gref = hwh_h.at[cid]
        else:
            ebase = (sid * 2 + cid) * ept
            gref = hwh_h.at[0]
        rbase = sid * ROWS_PT
        bufs = (buf0, buf1, buf2)
        dbs = (db0, db1, db2)
        sgs = (sg0, sg1, sg2)
        sss = (ss0, ss1, ss2)

        pltpu.sync_copy(degpart_h.at[0], dis_v)
        pltpu.sync_copy(bch_h.at[cid], bc_v)

        for piece in range(NPAD // npc):
            pltpu.sync_copy(degpart_h.at[1].at[pl.ds(piece * npc, npc)], tmp_v)

            def _sum_chunk(i, _):
                o = piece * npc + i * L
                d = dis_v[pl.ds(o, L)] + tmp_v[pl.ds(i * L, L)]
                dis_v[pl.ds(o, L)] = _rsqrt16(d)
                return _
            lax.fori_loop(0, npc // L, _sum_chunk, None)

        if feature_split:
            cz = 1.0
        else:
            cz = jnp.where(cid == 0, 1.0, 0.0)
        for b in range(ROWS_PT // CH):
            rows = rbase + b * CH
            pltpu.sync_copy(gref.at[pl.ds(rows, CH)], buf0)

            def _init_row(r, _):
                g = _bcast(dis_v, rows + r)
                rec = g * g * cz
                for j in range(width // L):
                    buf0[r, pl.ds(j * L, L)] = buf0[r, pl.ds(j * L, L)] * rec
                return _
            lax.fori_loop(0, CH, _init_row, None, unroll=2)
            pltpu.sync_copy(buf0, acc_sp.at[pl.ds(rows, CH)])
        plsc.subcore_barrier()

        def _start_g(k, ph):
            pltpu.async_copy(gref.at[src_v.at[pl.ds(k * CH, CH)]],
                             bufs[ph], sgs[ph])

        def _wait_g(ph):
            pltpu.make_async_copy(gref.at[pl.ds(0, CH)], bufs[ph],
                                  sgs[ph]).wait()

        def _start_s(ph):
            pltpu.async_copy(bufs[ph], acc_sp.at[dbs[ph].at[0]],
                             sss[ph], add=True)

        def _wait_s(ph):
            pltpu.make_async_copy(bufs[ph], acc_sp.at[pl.ds(0, CH)],
                                  sss[ph]).wait()

        def _step(k, ph, wait_s_prev, start_next):
            _wait_g(ph)
            if wait_s_prev:
                _wait_s((ph + 2) % 3)
            if start_next:
                _start_g(k + 1, (ph + 1) % 3)
            buf = bufs[ph]
            db = dbs[ph]
            for j in range(CH // L):
                o = k * CH + j * L
                s16 = src_v[pl.ds(o, L)]
                d16 = dst_v[pl.ds(o, L)]
                n16 = (plsc.load_gather(dis_v, [s16]) * ew_v[pl.ds(o, L)]
                       * plsc.load_gather(dis_v, [d16]))
                normbuf[pl.ds(j * L, L)] = n16
                db[0, pl.ds(j * L, L)] = d16

            def _scale_row(r, _):
                nb = _bcast(normbuf, r)
                for j in range(width // L):
                    buf[r, pl.ds(j * L, L)] = buf[r, pl.ds(j * L, L)] * nb
                return _
            lax.fori_loop(0, CH, _scale_row, None, unroll=2)
            _start_s(ph)

        for sp in range(nsp):
            sbase = ebase + sp * sph
            pltpu.sync_copy(src_h.at[pl.ds(sbase, sph)], src_v)
            pltpu.sync_copy(dst_h.at[pl.ds(sbase, sph)], dst_v)
            pltpu.sync_copy(ew_h.at[pl.ds(sbase, sph)], ew_v)
            _start_g(0, 0)
            _step(0, 0, False, True)
            _step(1, 1, True, True)
            _step(2, 2, True, True)

            def _group(g, _):
                _step(g * 3 + 0, 0, True, True)
                _step(g * 3 + 1, 1, True, True)
                _step(g * 3 + 2, 2, True, True)
                return _
            lax.fori_loop(1, ngr - 1, _group, None)
            _step(nch - 3, 0, True, True)
            _step(nch - 2, 1, True, True)
            _step(nch - 1, 2, True, False)
            _wait_s(2)
        plsc.subcore_barrier()

        for b in range(ROWS_PT // CH):
            rows = rbase + b * CH
            pltpu.sync_copy(acc_sp.at[pl.ds(rows, CH)], buf0)
            if feature_split:
                def _out_row(r, _):
                    for j in range(width // L):
                        v = buf0[r, pl.ds(j * L, L)] + bc_v[pl.ds(j * L, L)]
                        v = jnp.maximum(v, 0.0)
                        buf0[r, pl.ds(j * L, L)] = v
                    return _
                lax.fori_loop(0, CH, _out_row, None, unroll=2)
            pltpu.sync_copy(buf0, out_h.at[cid].at[pl.ds(rows, CH)])

    return agg


def _make_final():
    fch = 96
    nch = EPW // fch

    @functools.partial(
        pl.kernel,
        out_type=_f32((EPAD, 64)),
        mesh=_mesh(),
        compiler_params=_SC_PARAMS,
        scratch_types=dict(
            d0_v=pltpu.VMEM((EPW,), jnp.int32),
            d1_v=pltpu.VMEM((EPW,), jnp.int32),
            a0=pltpu.VMEM((fch, 128), jnp.float32),
            a1=pltpu.VMEM((fch, 128), jnp.float32),
            b0=pltpu.VMEM((fch, 128), jnp.float32),
            b1=pltpu.VMEM((fch, 128), jnp.float32),
            q0=pltpu.VMEM((fch, 64), jnp.float32),
            q1=pltpu.VMEM((fch, 64), jnp.float32),
            sa0=pltpu.SemaphoreType.DMA,
            sa1=pltpu.SemaphoreType.DMA,
            sb0=pltpu.SemaphoreType.DMA,
            sb1=pltpu.SemaphoreType.DMA,
            sw0=pltpu.SemaphoreType.DMA,
            sw1=pltpu.SemaphoreType.DMA,
        ),
    )
    def final(p_h, d0_h, d1_h, q_h, d0_v, d1_v, a0, a1, b0, b1, q0, q1,
              sa0, sa1, sb0, sb1, sw0, sw1):
        cid = lax.axis_index("c")
        sid = lax.axis_index("s")
        base = (sid * 2 + cid) * EPW
        abufs = (a0, a1)
        bbufs = (b0, b1)
        qbufs = (q0, q1)
        sas = (sa0, sa1)
        sbs = (sb0, sb1)
        sws = (sw0, sw1)

        pltpu.sync_copy(d0_h.at[pl.ds(base, EPW)], d0_v)
        pltpu.sync_copy(d1_h.at[pl.ds(base, EPW)], d1_v)

        def _start_g(k, ph):
            pltpu.async_copy(p_h.at[d0_v.at[pl.ds(k * fch, fch)]],
                             abufs[ph], sas[ph])
            pltpu.async_copy(p_h.at[d1_v.at[pl.ds(k * fch, fch)]],
                             bbufs[ph], sbs[ph])

        def _wait_g(ph):
            pltpu.make_async_copy(p_h.at[pl.ds(0, fch)], abufs[ph],
                                  sas[ph]).wait()
            pltpu.make_async_copy(p_h.at[pl.ds(0, fch)], bbufs[ph],
                                  sbs[ph]).wait()

        def _wait_w(ph):
            pltpu.make_async_copy(qbufs[ph], q_h.at[pl.ds(0, fch)],
                                  sws[ph]).wait()

        def _step(k, ph, wait_w_prev, start_next):
            _wait_g(ph)
            if start_next:
                _start_g(k + 1, 1 - ph)
            if wait_w_prev:
                _wait_w(1 - ph)
            av = abufs[ph]
            bv = bbufs[ph]
            qv = qbufs[ph]

            def _sub_row(r, _):
                for j in range(4):
                    qv[r, pl.ds(j * L, L)] = (av[r, pl.ds(j * L, L)]
                                              - bv[r, pl.ds(j * L, L)])
                return _
            lax.fori_loop(0, fch, _sub_row, None, unroll=2)
            pltpu.async_copy(qv, q_h.at[pl.ds(base + k * fch, fch)], sws[ph])

        _start_g(0, 0)
        _step(0, 0, False, True)

        def _group(g, _):
            _step(g * 2 - 1, 1, True, True)
            _step(g * 2, 0, True, True)
            return _
        lax.fori_loop(1, nch // 2, _group, None)
        _step(nch - 1, 1, True, False)
        _wait_w(1)

    return final


_edge1 = _make_edge(True)
_edge2 = _make_edge(False)
_agg1 = _make_agg(True)
_agg2 = _make_agg(False)
_final = _make_final()


def kernel(x, coords, edge_index, data_edge_index, Wa, ba, Wg, bg, Waf, baf,
           Wc1, bc1, Wm1, bm1, Wc2, bc2, Wf1, bf1, Wf2, bf2):
    f32 = jnp.float32
    xp = jnp.zeros((NPAD, 512), f32).at[:N].set(x)
    cp = jnp.zeros((NPAD, 4), f32).at[:N].set(coords)
    pad_idx = (N + jnp.arange(EPAD - E, dtype=jnp.int32) % (NPAD - N))
    srcp = jnp.concatenate([edge_index[0], pad_idx])
    dstp = jnp.concatenate([edge_index[1], pad_idx])
    d0p = jnp.concatenate([data_edge_index[0], pad_idx])
    d1p = jnp.concatenate([data_edge_index[1], pad_idx])

    grid = NPAD // RB
    wnode = Wa[0].reshape(2, 512).T
    wgnode = Wg[0].reshape(2, 4).T
    hwh1, ns = pl.pallas_call(
        _tc1_body,
        grid=(grid,),
        in_specs=[
            pl.BlockSpec((RB, 512), lambda i: (i, 0)),
            pl.BlockSpec((RB, 4), lambda i: (i, 0)),
            pl.BlockSpec((512, 256), lambda i: (0, 0)),
            pl.BlockSpec((512, 2), lambda i: (0, 0)),
            pl.BlockSpec((4, 2), lambda i: (0, 0)),
        ],
        out_specs=[
            pl.BlockSpec((2, RB, 128), lambda i: (0, i, 0)),
            pl.BlockSpec((RB, 4), lambda i: (i, 0)),
        ],
        out_shape=[_f32((2, NPAD, 128)), _f32((NPAD, 4))],
    )(xp, cp, Wc1.T, wnode, wgnode)

    z16 = jnp.zeros((L,), f32)
    consts1 = jnp.stack([
        z16 + ba[0], z16 + bg[0], z16 + baf[0],
        z16 + Waf[0, 0], z16 + Waf[0, 1],
        z16, z16, z16])
    ew1, degpart1 = _edge1(ns.reshape(-1), srcp, dstp, consts1)
    h = _agg1(srcp, dstp, ew1, degpart1, hwh1, bc1.reshape(2, 128))

    wm = Wm1[0].reshape(2, 256).T
    hw2, ms = pl.pallas_call(
        _tc2_body,
        grid=(grid,),
        in_specs=[
            pl.BlockSpec((1, RB, 128), lambda i: (0, i, 0)),
            pl.BlockSpec((1, RB, 128), lambda i: (1, i, 0)),
            pl.BlockSpec((128, 128), lambda i: (0, 0)),
            pl.BlockSpec((128, 128), lambda i: (0, 0)),
            pl.BlockSpec((256, 2), lambda i: (0, 0)),
        ],
        out_specs=[
            pl.BlockSpec((RB, 128), lambda i: (i, 0)),
            pl.BlockSpec((RB, 2), lambda i: (i, 0)),
        ],
        out_shape=[_f32((NPAD, 128)), _f32((NPAD, 2))],
    )(h, h, Wc2[:, :128].T, Wc2[:, 128:].T, wm)

    consts2 = jnp.stack([z16 + bm1[0]] + [z16] * 7)
    ew2, degpart2 = _edge2(ms.reshape(-1), srcp, dstp, consts2)
    part = _agg2(srcp, dstp, ew2, degpart2, hw2.reshape(1, NPAD, 128),
                 jnp.zeros((2, 128), f32))

    p = pl.pallas_call(
        _tc3_body,
        grid=(grid,),
        in_specs=[
            pl.BlockSpec((1, RB, 128), lambda i: (0, i, 0)),
            pl.BlockSpec((1, RB, 128), lambda i: (1, i, 0)),
            pl.BlockSpec((1, 128), lambda i: (0, 0)),
            pl.BlockSpec((128, 64), lambda i: (0, 0)),
        ],
        out_specs=pl.BlockSpec((RB, 128), lambda i: (i, 0)),
        out_shape=_f32((NPAD, 128)),
    )(part, part, bc2.reshape(1, 128), Wf1.T)

    q = _final(p, d0p, d1p)

    prob = pl.pallas_call(
        _tc4_body,
        grid=(EPAD // RB,),
        in_specs=[
            pl.BlockSpec((RB, 64), lambda i: (i, 0)),
            pl.BlockSpec((1, 64), lambda i: (0, 0)),
            pl.BlockSpec((64, 1), lambda i: (0, 0)),
            pl.BlockSpec((1, 1), lambda i: (0, 0)),
        ],
        out_specs=pl.BlockSpec((RB, 1), lambda i: (i, 0)),
        out_shape=_f32((EPAD, 1)),
    )(q, bf1.reshape(1, 64), Wf2.reshape(64, 1), bf2.reshape(1, 1))

    return prob[:E]


# speedup vs baseline: 10.9379x; 1.0159x over previous
import functools
import jax
import jax.numpy as jnp
from jax import lax
from jax.experimental import pallas as pl
from jax.experimental.pallas import tpu as pltpu
from jax.experimental.pallas import tpu_sc as plsc

N = 10000
E = 150000
NPAD = 10240
EPAD = 153600
EPT = EPAD // 16
EPW = EPAD // 32
ROWS_PT = NPAD // 16
RB = 2048
L = 16
CH = 64

_SC_PARAMS = pltpu.CompilerParams(needs_layout_passes=False)


def _f32(shape):
    return jax.ShapeDtypeStruct(shape, jnp.float32)


def _mesh():
    return plsc.VectorSubcoreMesh(core_axis_name="c", subcore_axis_name="s")


def _tc1_body(x_ref, c_ref, wc1t_ref, wn_ref, wg_ref, hwh_ref, ns_ref):
    xb = x_ref[...]
    hw = jnp.dot(xb, wc1t_ref[...], preferred_element_type=jnp.float32)
    hwh_ref[0] = hw[:, :128]
    hwh_ref[1] = hw[:, 128:]
    ns_ref[...] = jnp.concatenate(
        [jnp.dot(xb, wn_ref[...], preferred_element_type=jnp.float32),
         jnp.dot(c_ref[...], wg_ref[...], preferred_element_type=jnp.float32)],
        axis=1)


def _tc2_body(h0_ref, h1_ref, w2a_ref, w2b_ref, wm_ref, hw_ref, ms_ref):
    h0 = h0_ref[0]
    h1 = h1_ref[0]
    hw_ref[...] = (jnp.dot(h0, w2a_ref[...], preferred_element_type=jnp.float32)
                   + jnp.dot(h1, w2b_ref[...], preferred_element_type=jnp.float32))
    ms_ref[...] = (jnp.dot(h0, wm_ref[...][:128], preferred_element_type=jnp.float32)
                   + jnp.dot(h1, wm_ref[...][128:], preferred_element_type=jnp.float32))


def _tc3_body(o0_ref, o1_ref, bc2_ref, f1t_ref, p_ref):
    out = o0_ref[0] + o1_ref[0] + bc2_ref[...]
    p = jnp.dot(out, f1t_ref[...], preferred_element_type=jnp.float32)
    p_ref[...] = jnp.concatenate([p, jnp.zeros_like(p)], axis=1)


def _tc4_body(q_ref, bf1_ref, wf2_ref, bf2_ref, out_ref):
    hid = jnp.maximum(q_ref[...] + bf1_ref[...], 0.0)
    logit = jnp.dot(hid, wf2_ref[...], preferred_element_type=jnp.float32) + bf2_ref[0, 0]
    out_ref[...] = jax.nn.sigmoid(logit)


def _rsqrt16(d):
    i = plsc.bitcast(d, jnp.int32)
    i = jnp.int32(0x5F3759DF) - (i >> 1)
    y = plsc.bitcast(i, jnp.float32)
    for _ in range(3):
        y = y * (1.5 - 0.5 * d * y * y)
    return y


def _bcast(ref, r):
    return plsc.load_gather(ref, [jnp.full((L,), r, jnp.int32)])


_GDN = lax.GatherDimensionNumbers(
    offset_dims=(), collapsed_slice_dims=(0,), start_index_map=(0,))


def _vbcast(vec16, r):
    idx = jnp.full((L, 1), r, jnp.int32)
    return lax.gather(vec16, idx, _GDN, (1,),
                      mode=lax.GatherScatterMode.PROMISE_IN_BOUNDS)


def _make_edge(two_layer):
    tc = 4 if two_layer else 2
    nch = EPW // 128
    tail = EPW - nch * 128

    @functools.partial(
        pl.kernel,
        out_type=(_f32((EPAD,)), _f32((2, NPAD))),
        mesh=_mesh(),
        compiler_params=_SC_PARAMS,
        scratch_types=dict(
            tbl_v=pltpu.VMEM((NPAD * tc,), jnp.float32),
            src_v=pltpu.VMEM((EPW,), jnp.int32),
            dst_v=pltpu.VMEM((EPW,), jnp.int32),
            ew_v=pltpu.VMEM((EPW,), jnp.float32),
            dstbuf=pltpu.VMEM((1, 128), jnp.int32),
            dstbuf_t=pltpu.VMEM((1, tail), jnp.int32),
            slice_v=pltpu.VMEM((ROWS_PT,), jnp.float32),
            consts_v=pltpu.VMEM((8, L), jnp.float32),
            deg_sp=pltpu.VMEM_SHARED((NPAD,), jnp.float32),
        ),
    )
    def edge(tbl_h, src_h, dst_h, consts_h, ew_h, degpart_h,
             tbl_v, src_v, dst_v, ew_v, dstbuf, dstbuf_t, slice_v,
             consts_v, deg_sp):
        cid = lax.axis_index("c")
        sid = lax.axis_index("s")
        ebase = (sid * 2 + cid) * EPW
        rbase = sid * ROWS_PT

        pltpu.sync_copy(tbl_h, tbl_v)
        pltpu.sync_copy(src_h.at[pl.ds(ebase, EPW)], src_v)
        pltpu.sync_copy(dst_h.at[pl.ds(ebase, EPW)], dst_v)
        pltpu.sync_copy(consts_h, consts_v)

        init = jnp.where(cid == 0, 1.0, 0.0)

        def _fill(i, _):
            slice_v[pl.ds(i * L, L)] = jnp.full((L,), init, jnp.float32)
            return _
        lax.fori_loop(0, ROWS_PT // L, _fill, None)
        pltpu.sync_copy(slice_v, deg_sp.at[pl.ds(rbase, ROWS_PT)])
        plsc.subcore_barrier()

        c0 = consts_v[0]
        c1 = consts_v[1]
        c2 = consts_v[2]
        c3 = consts_v[3]
        c4 = consts_v[4]

        def _ew16(o):
            d16r = dst_v[pl.ds(o, L)]
            s16 = src_v[pl.ds(o, L)] * tc
            d16 = d16r * tc
            if two_layer:
                sA = plsc.load_gather(tbl_v, [s16])
                dA = plsc.load_gather(tbl_v, [d16 + 1])
                sG = plsc.load_gather(tbl_v, [s16 + 2])
                dG = plsc.load_gather(tbl_v, [d16 + 3])
                x1 = jnp.maximum(sA + dA + c0, 0.0)
                x2 = jnp.maximum(sG + dG + c1, 0.0)
                ew = jnp.maximum(c3 * x1 + c4 * x2 + c2, 0.0)
            else:
                sA = plsc.load_gather(tbl_v, [s16])
                dA = plsc.load_gather(tbl_v, [d16 + 1])
                ew = jnp.maximum(sA + dA + c0, 0.0)
            gid = ebase + o + lax.iota(jnp.int32, L)
            ew = jnp.where(gid < E, ew, 0.0)
            ew_v[pl.ds(o, L)] = ew
            return d16r

        def _chunk(k, _):
            eb = k * 128
            for j in range(8):
                dstbuf[0, pl.ds(j * L, L)] = _ew16(eb + j * L)
            pltpu.sync_copy(ew_v.at[pl.ds(eb, 128)],
                            deg_sp.at[dstbuf.at[0]], add=True)
            return _
        lax.fori_loop(0, nch, _chunk, None)
        for j in range(tail // L):
            dstbuf_t[0, pl.ds(j * L, L)] = _ew16(nch * 128 + j * L)
        pltpu.sync_copy(ew_v.at[pl.ds(nch * 128, tail)],
                        deg_sp.at[dstbuf_t.at[0]], add=True)

        pltpu.sync_copy(ew_v, ew_h.at[pl.ds(ebase, EPW)])
        plsc.subcore_barrier()

        pltpu.sync_copy(deg_sp.at[pl.ds(rbase, ROWS_PT)], slice_v)
        pltpu.sync_copy(slice_v, degpart_h.at[cid].at[pl.ds(rbase, ROWS_PT)])

    return edge


def _make_agg(feature_split):
    ept = EPT if feature_split else EPW
    sph = 1920 if feature_split else 960
    nsp = ept // sph
    nch = sph // CH
    ngr = nch // 3
    npc = 2560
    width = 128

    @functools.partial(
        pl.kernel,
        out_type=_f32((2, NPAD, width)),
        mesh=_mesh(),
        compiler_params=_SC_PARAMS,
        scratch_types=dict(
            src_v=pltpu.VMEM((sph,), jnp.int32),
            dst_v=pltpu.VMEM((sph,), jnp.int32),
            ew_v=pltpu.VMEM((sph,), jnp.float32),
            dis_v=pltpu.VMEM((NPAD,), jnp.float32),
            tmp_v=pltpu.VMEM((npc,), jnp.float32),
            buf0=pltpu.VMEM((CH, width), jnp.float32),
            buf1=pltpu.VMEM((CH, width), jnp.float32),
            buf2=pltpu.VMEM((CH, width), jnp.float32),
            db0=pltpu.VMEM((1, CH), jnp.int32),
            db1=pltpu.VMEM((1, CH), jnp.int32),
            db2=pltpu.VMEM((1, CH), jnp.int32),
            bc_v=pltpu.VMEM((width,), jnp.float32),
            acc_sp=pltpu.VMEM_SHARED((NPAD, width), jnp.float32),
            sg0=pltpu.SemaphoreType.DMA,
            sg1=pltpu.SemaphoreType.DMA,
            sg2=pltpu.SemaphoreType.DMA,
            ss0=pltpu.SemaphoreType.DMA,
            ss1=pltpu.SemaphoreType.DMA,
            ss2=pltpu.SemaphoreType.DMA,
        ),
    )
    def agg(src_h, dst_h, ew_h, degpart_h, hwh_h, bch_h, out_h,
            src_v, dst_v, ew_v, dis_v, tmp_v, buf0, buf1, buf2,
            db0, db1, db2, bc_v, acc_sp, sg0, sg1, sg2,
            ss0, ss1, ss2):
        cid = lax.axis_index("c")
        sid = lax.axis_index("s")
        if feature_split:
            ebase = sid * ept
            gref = hwh_h.at[cid]
        else:
            ebase = (sid * 2 + cid) * ept
            gref = hwh_h.at[0]
        rbase = sid * ROWS_PT
        bufs = (buf0, buf1, buf2)
        dbs = (db0, db1, db2)
        sgs = (sg0, sg1, sg2)
        sss = (ss0, ss1, ss2)

        pltpu.sync_copy(degpart_h.at[0], dis_v)
        pltpu.sync_copy(bch_h.at[cid], bc_v)

        for piece in range(NPAD // npc):
            pltpu.sync_copy(degpart_h.at[1].at[pl.ds(piece * npc, npc)], tmp_v)

            def _sum_chunk(i, _):
                o = piece * npc + i * L
                d = dis_v[pl.ds(o, L)] + tmp_v[pl.ds(i * L, L)]
                dis_v[pl.ds(o, L)] = _rsqrt16(d)
                return _
            lax.fori_loop(0, npc // L, _sum_chunk, None)

        if feature_split:
            cz = 1.0
        else:
            cz = jnp.where(cid == 0, 1.0, 0.0)
        for b in range(ROWS_PT // CH):
            rows = rbase + b * CH
            pltpu.sync_copy(gref.at[pl.ds(rows, CH)], buf0)
            for g16 in range(CH // L):
                d16 = dis_v[pl.ds(rows + g16 * L, L)]
                rec16 = d16 * d16 * cz

                def _init_row(r, _):
                    rb = _vbcast(rec16, r)
                    row = g16 * L + r
                    for j in range(width // L):
                        buf0[row, pl.ds(j * L, L)] = (
                            buf0[row, pl.ds(j * L, L)] * rb)
                    return _
                lax.fori_loop(0, L, _init_row, None, unroll=4)
            pltpu.sync_copy(buf0, acc_sp.at[pl.ds(rows, CH)])
        plsc.subcore_barrier()

        def _start_g(k, ph):
            pltpu.async_copy(gref.at[src_v.at[pl.ds(k * CH, CH)]],
                             bufs[ph], sgs[ph])

        def _wait_g(ph):
            pltpu.make_async_copy(gref.at[pl.ds(0, CH)], bufs[ph],
                                  sgs[ph]).wait()

        def _start_s(ph):
            pltpu.async_copy(bufs[ph], acc_sp.at[dbs[ph].at[0]],
                             sss[ph], add=True)

        def _wait_s(ph):
            pltpu.make_async_copy(bufs[ph], acc_sp.at[pl.ds(0, CH)],
                                  sss[ph]).wait()

        def _step(k, ph, wait_s_prev, start_next):
            _wait_g(ph)
            if wait_s_prev:
                _wait_s((ph + 2) % 3)
            if start_next:
                _start_g(k + 1, (ph + 1) % 3)
            buf = bufs[ph]
            db = dbs[ph]
            for j in range(CH // L):
                o = k * CH + j * L
                s16 = src_v[pl.ds(o, L)]
                d16 = dst_v[pl.ds(o, L)]
                n16 = (plsc.load_gather(dis_v, [s16]) * ew_v[pl.ds(o, L)]
                       * plsc.load_gather(dis_v, [d16]))
                db[0, pl.ds(j * L, L)] = d16

                def _scale_row(r, _):
                    nb = _vbcast(n16, r)
                    row = j * L + r
                    for jj in range(width // L):
                        buf[row, pl.ds(jj * L, L)] = (
                            buf[row, pl.ds(jj * L, L)] * nb)
                    return _
                lax.fori_loop(0, L, _scale_row, None, unroll=4)
            _start_s(ph)

        def _subphase(sp, _):
            sbase = ebase + sp * sph
            pltpu.sync_copy(src_h.at[pl.ds(sbase, sph)], src_v)
            pltpu.sync_copy(dst_h.at[pl.ds(sbase, sph)], dst_v)
            pltpu.sync_copy(ew_h.at[pl.ds(sbase, sph)], ew_v)
            _start_g(0, 0)
            _step(0, 0, False, True)
            _step(1, 1, True, True)
            _step(2, 2, True, True)

            def _group(g, _):
                _step(g * 3 + 0, 0, True, True)
                _step(g * 3 + 1, 1, True, True)
                _step(g * 3 + 2, 2, True, True)
                return _
            lax.fori_loop(1, ngr - 1, _group, None)
            _step(nch - 3, 0, True, True)
            _step(nch - 2, 1, True, True)
            _step(nch - 1, 2, True, False)
            _wait_s(2)
            return _
        lax.fori_loop(0, nsp, _subphase, None)
        plsc.subcore_barrier()

        for b in range(ROWS_PT // CH):
            rows = rbase + b * CH
            pltpu.sync_copy(acc_sp.at[pl.ds(rows, CH)], buf0)
            if feature_split:
                def _out_row(r, _):
                    for j in range(width // L):
                        v = buf0[r, pl.ds(j * L, L)] + bc_v[pl.ds(j * L, L)]
                        v = jnp.maximum(v, 0.0)
                        buf0[r, pl.ds(j * L, L)] = v
                    return _
                lax.fori_loop(0, CH, _out_row, None, unroll=2)
            pltpu.sync_copy(buf0, out_h.at[cid].at[pl.ds(rows, CH)])

    return agg


def _make_final():
    fch = 96
    nch = EPW // fch

    @functools.partial(
        pl.kernel,
        out_type=_f32((EPAD, 64)),
        mesh=_mesh(),
        compiler_params=_SC_PARAMS,
        scratch_types=dict(
            d0_v=pltpu.VMEM((EPW,), jnp.int32),
            d1_v=pltpu.VMEM((EPW,), jnp.int32),
            a0=pltpu.VMEM((fch, 128), jnp.float32),
            a1=pltpu.VMEM((fch, 128), jnp.float32),
            b0=pltpu.VMEM((fch, 128), jnp.float32),
            b1=pltpu.VMEM((fch, 128), jnp.float32),
            q0=pltpu.VMEM((fch, 64), jnp.float32),
            q1=pltpu.VMEM((fch, 64), jnp.float32),
            sa0=pltpu.SemaphoreType.DMA,
            sa1=pltpu.SemaphoreType.DMA,
            sb0=pltpu.SemaphoreType.DMA,
            sb1=pltpu.SemaphoreType.DMA,
            sw0=pltpu.SemaphoreType.DMA,
            sw1=pltpu.SemaphoreType.DMA,
        ),
    )
    def final(p_h, d0_h, d1_h, q_h, d0_v, d1_v, a0, a1, b0, b1, q0, q1,
              sa0, sa1, sb0, sb1, sw0, sw1):
        cid = lax.axis_index("c")
        sid = lax.axis_index("s")
        base = (sid * 2 + cid) * EPW
        abufs = (a0, a1)
        bbufs = (b0, b1)
        qbufs = (q0, q1)
        sas = (sa0, sa1)
        sbs = (sb0, sb1)
        sws = (sw0, sw1)

        pltpu.sync_copy(d0_h.at[pl.ds(base, EPW)], d0_v)
        pltpu.sync_copy(d1_h.at[pl.ds(base, EPW)], d1_v)

        def _start_g(k, ph):
            pltpu.async_copy(p_h.at[d0_v.at[pl.ds(k * fch, fch)]],
                             abufs[ph], sas[ph])
            pltpu.async_copy(p_h.at[d1_v.at[pl.ds(k * fch, fch)]],
                             bbufs[ph], sbs[ph])

        def _wait_g(ph):
            pltpu.make_async_copy(p_h.at[pl.ds(0, fch)], abufs[ph],
                                  sas[ph]).wait()
            pltpu.make_async_copy(p_h.at[pl.ds(0, fch)], bbufs[ph],
                                  sbs[ph]).wait()

        def _wait_w(ph):
            pltpu.make_async_copy(qbufs[ph], q_h.at[pl.ds(0, fch)],
                                  sws[ph]).wait()

        def _step(k, ph, wait_w_prev, start_next):
            _wait_g(ph)
            if start_next:
                _start_g(k + 1, 1 - ph)
            if wait_w_prev:
                _wait_w(1 - ph)
            av = abufs[ph]
            bv = bbufs[ph]
            qv = qbufs[ph]

            def _sub_row(r, _):
                for j in range(4):
                    qv[r, pl.ds(j * L, L)] = (av[r, pl.ds(j * L, L)]
                                              - bv[r, pl.ds(j * L, L)])
                return _
            lax.fori_loop(0, fch, _sub_row, None, unroll=4)
            pltpu.async_copy(qv, q_h.at[pl.ds(base + k * fch, fch)], sws[ph])

        _start_g(0, 0)
        _step(0, 0, False, True)

        def _group(g, _):
            _step(g * 2 - 1, 1, True, True)
            _step(g * 2, 0, True, True)
            return _
        lax.fori_loop(1, nch // 2, _group, None)
        _step(nch - 1, 1, True, False)
        _wait_w(1)

    return final


_edge1 = _make_edge(True)
_edge2 = _make_edge(False)
_agg1 = _make_agg(True)
_agg2 = _make_agg(False)
_final = _make_final()


def kernel(x, coords, edge_index, data_edge_index, Wa, ba, Wg, bg, Waf, baf,
           Wc1, bc1, Wm1, bm1, Wc2, bc2, Wf1, bf1, Wf2, bf2):
    f32 = jnp.float32
    xp = jnp.zeros((NPAD, 512), f32).at[:N].set(x)
    cp = jnp.zeros((NPAD, 4), f32).at[:N].set(coords)
    pad_idx = (N + jnp.arange(EPAD - E, dtype=jnp.int32) % (NPAD - N))
    srcp = jnp.concatenate([edge_index[0], pad_idx])
    dstp = jnp.concatenate([edge_index[1], pad_idx])
    d0p = jnp.concatenate([data_edge_index[0], pad_idx])
    d1p = jnp.concatenate([data_edge_index[1], pad_idx])

    grid = NPAD // RB
    wnode = Wa[0].reshape(2, 512).T
    wgnode = Wg[0].reshape(2, 4).T
    hwh1, ns = pl.pallas_call(
        _tc1_body,
        grid=(grid,),
        in_specs=[
            pl.BlockSpec((RB, 512), lambda i: (i, 0)),
            pl.BlockSpec((RB, 4), lambda i: (i, 0)),
            pl.BlockSpec((512, 256), lambda i: (0, 0)),
            pl.BlockSpec((512, 2), lambda i: (0, 0)),
            pl.BlockSpec((4, 2), lambda i: (0, 0)),
        ],
        out_specs=[
            pl.BlockSpec((2, RB, 128), lambda i: (0, i, 0)),
            pl.BlockSpec((RB, 4), lambda i: (i, 0)),
        ],
        out_shape=[_f32((2, NPAD, 128)), _f32((NPAD, 4))],
    )(xp, cp, Wc1.T, wnode, wgnode)

    z16 = jnp.zeros((L,), f32)
    consts1 = jnp.stack([
        z16 + ba[0], z16 + bg[0], z16 + baf[0],
        z16 + Waf[0, 0], z16 + Waf[0, 1],
        z16, z16, z16])
    ew1, degpart1 = _edge1(ns.reshape(-1), srcp, dstp, consts1)
    h = _agg1(srcp, dstp, ew1, degpart1, hwh1, bc1.reshape(2, 128))

    wm = Wm1[0].reshape(2, 256).T
    hw2, ms = pl.pallas_call(
        _tc2_body,
        grid=(grid,),
        in_specs=[
            pl.BlockSpec((1, RB, 128), lambda i: (0, i, 0)),
            pl.BlockSpec((1, RB, 128), lambda i: (1, i, 0)),
            pl.BlockSpec((128, 128), lambda i: (0, 0)),
            pl.BlockSpec((128, 128), lambda i: (0, 0)),
            pl.BlockSpec((256, 2), lambda i: (0, 0)),
        ],
        out_specs=[
            pl.BlockSpec((RB, 128), lambda i: (i, 0)),
            pl.BlockSpec((RB, 2), lambda i: (i, 0)),
        ],
        out_shape=[_f32((NPAD, 128)), _f32((NPAD, 2))],
    )(h, h, Wc2[:, :128].T, Wc2[:, 128:].T, wm)

    consts2 = jnp.stack([z16 + bm1[0]] + [z16] * 7)
    ew2, degpart2 = _edge2(ms.reshape(-1), srcp, dstp, consts2)
    part = _agg2(srcp, dstp, ew2, degpart2, hw2.reshape(1, NPAD, 128),
                 jnp.zeros((2, 128), f32))

    p = pl.pallas_call(
        _tc3_body,
        grid=(grid,),
        in_specs=[
            pl.BlockSpec((1, RB, 128), lambda i: (0, i, 0)),
            pl.BlockSpec((1, RB, 128), lambda i: (1, i, 0)),
            pl.BlockSpec((1, 128), lambda i: (0, 0)),
            pl.BlockSpec((128, 64), lambda i: (0, 0)),
        ],
        out_specs=pl.BlockSpec((RB, 128), lambda i: (i, 0)),
        out_shape=_f32((NPAD, 128)),
    )(part, part, bc2.reshape(1, 128), Wf1.T)

    q = _final(p, d0p, d1p)

    prob = pl.pallas_call(
        _tc4_body,
        grid=(EPAD // RB,),
        in_specs=[
            pl.BlockSpec((RB, 64), lambda i: (i, 0)),
            pl.BlockSpec((1, 64), lambda i: (0, 0)),
            pl.BlockSpec((64, 1), lambda i: (0, 0)),
            pl.BlockSpec((1, 1), lambda i: (0, 0)),
        ],
        out_specs=pl.BlockSpec((RB, 1), lambda i: (i, 0)),
        out_shape=_f32((EPAD, 1)),
    )(q, bf1.reshape(1, 64), Wf2.reshape(64, 1), bf2.reshape(1, 1))

    return prob[:E]


# speedup vs baseline: 12.1640x; 1.1121x over previous
import functools
import jax
import jax.numpy as jnp
from jax import lax
from jax.experimental import pallas as pl
from jax.experimental.pallas import tpu as pltpu
from jax.experimental.pallas import tpu_sc as plsc

N = 10000
E = 150000
NPAD = 10240
EPAD = 153600
EPT = EPAD // 16
EPW = EPAD // 32
ROWS_PT = NPAD // 16
RB = 2048
L = 16
CH = 64

_SC_PARAMS = pltpu.CompilerParams(needs_layout_passes=False)


def _f32(shape):
    return jax.ShapeDtypeStruct(shape, jnp.float32)


def _mesh():
    return plsc.VectorSubcoreMesh(core_axis_name="c", subcore_axis_name="s")


def _tc1_body(x_ref, c_ref, wc1t_ref, wn_ref, wg_ref, hwh_ref, ns_ref):
    xb = x_ref[...]
    hw = jnp.dot(xb, wc1t_ref[...], preferred_element_type=jnp.float32)
    hwh_ref[0] = hw[:, :128]
    hwh_ref[1] = hw[:, 128:]
    ns_ref[...] = jnp.concatenate(
        [jnp.dot(xb, wn_ref[...], preferred_element_type=jnp.float32),
         jnp.dot(c_ref[...], wg_ref[...], preferred_element_type=jnp.float32)],
        axis=1)


def _tc2_body(h0_ref, h1_ref, w2a_ref, w2b_ref, wm_ref, hw_ref, ms_ref):
    h0 = h0_ref[0]
    h1 = h1_ref[0]
    hw_ref[...] = (jnp.dot(h0, w2a_ref[...], preferred_element_type=jnp.float32)
                   + jnp.dot(h1, w2b_ref[...], preferred_element_type=jnp.float32))
    ms_ref[...] = (jnp.dot(h0, wm_ref[...][:128], preferred_element_type=jnp.float32)
                   + jnp.dot(h1, wm_ref[...][128:], preferred_element_type=jnp.float32))


def _tc3_body(o0_ref, o1_ref, bc2_ref, f1t_ref, p_ref):
    out = o0_ref[0] + o1_ref[0] + bc2_ref[...]
    p = jnp.dot(out, f1t_ref[...], preferred_element_type=jnp.float32)
    p_ref[...] = jnp.concatenate([p, jnp.zeros_like(p)], axis=1)


def _tc4_body(q_ref, bf1_ref, wf2_ref, bf2_ref, out_ref):
    hid = jnp.maximum(q_ref[...] + bf1_ref[...], 0.0)
    logit = jnp.dot(hid, wf2_ref[...], preferred_element_type=jnp.float32) + bf2_ref[0, 0]
    out_ref[...] = jax.nn.sigmoid(logit)


def _rsqrt16(d):
    i = plsc.bitcast(d, jnp.int32)
    i = jnp.int32(0x5F3759DF) - (i >> 1)
    y = plsc.bitcast(i, jnp.float32)
    for _ in range(3):
        y = y * (1.5 - 0.5 * d * y * y)
    return y


def _bcast(ref, r):
    return plsc.load_gather(ref, [jnp.full((L,), r, jnp.int32)])


_GDN = lax.GatherDimensionNumbers(
    offset_dims=(), collapsed_slice_dims=(0,), start_index_map=(0,))


def _vbcast(vec16, r):
    idx = jnp.full((L, 1), r, jnp.int32)
    return lax.gather(vec16, idx, _GDN, (1,),
                      mode=lax.GatherScatterMode.PROMISE_IN_BOUNDS)


def _make_edge(two_layer):
    tc = 4 if two_layer else 2
    nch = EPW // 128
    tail = EPW - nch * 128

    @functools.partial(
        pl.kernel,
        out_type=(_f32((EPAD,)), _f32((2, NPAD))),
        mesh=_mesh(),
        compiler_params=_SC_PARAMS,
        scratch_types=dict(
            tbl_v=pltpu.VMEM((NPAD * tc,), jnp.float32),
            src_v=pltpu.VMEM((EPW,), jnp.int32),
            dst_v=pltpu.VMEM((EPW,), jnp.int32),
            ew_v=pltpu.VMEM((EPW,), jnp.float32),
            dstbuf=pltpu.VMEM((1, 128), jnp.int32),
            dstbuf_t=pltpu.VMEM((1, tail), jnp.int32),
            slice_v=pltpu.VMEM((ROWS_PT,), jnp.float32),
            consts_v=pltpu.VMEM((8, L), jnp.float32),
            deg_sp=pltpu.VMEM_SHARED((NPAD,), jnp.float32),
        ),
    )
    def edge(tbl_h, src_h, dst_h, consts_h, ew_h, degpart_h,
             tbl_v, src_v, dst_v, ew_v, dstbuf, dstbuf_t, slice_v,
             consts_v, deg_sp):
        cid = lax.axis_index("c")
        sid = lax.axis_index("s")
        ebase = (sid * 2 + cid) * EPW
        rbase = sid * ROWS_PT

        pltpu.sync_copy(tbl_h, tbl_v)
        pltpu.sync_copy(src_h.at[pl.ds(ebase, EPW)], src_v)
        pltpu.sync_copy(dst_h.at[pl.ds(ebase, EPW)], dst_v)
        pltpu.sync_copy(consts_h, consts_v)

        init = jnp.where(cid == 0, 1.0, 0.0)

        def _fill(i, _):
            slice_v[pl.ds(i * L, L)] = jnp.full((L,), init, jnp.float32)
            return _
        lax.fori_loop(0, ROWS_PT // L, _fill, None)
        pltpu.sync_copy(slice_v, deg_sp.at[pl.ds(rbase, ROWS_PT)])
        plsc.subcore_barrier()

        c0 = consts_v[0]
        c1 = consts_v[1]
        c2 = consts_v[2]
        c3 = consts_v[3]
        c4 = consts_v[4]

        def _ew16(o):
            d16r = dst_v[pl.ds(o, L)]
            s16 = src_v[pl.ds(o, L)] * tc
            d16 = d16r * tc
            if two_layer:
                sA = plsc.load_gather(tbl_v, [s16])
                dA = plsc.load_gather(tbl_v, [d16 + 1])
                sG = plsc.load_gather(tbl_v, [s16 + 2])
                dG = plsc.load_gather(tbl_v, [d16 + 3])
                x1 = jnp.maximum(sA + dA + c0, 0.0)
                x2 = jnp.maximum(sG + dG + c1, 0.0)
                ew = jnp.maximum(c3 * x1 + c4 * x2 + c2, 0.0)
            else:
                sA = plsc.load_gather(tbl_v, [s16])
                dA = plsc.load_gather(tbl_v, [d16 + 1])
                ew = jnp.maximum(sA + dA + c0, 0.0)
            gid = ebase + o + lax.iota(jnp.int32, L)
            ew = jnp.where(gid < E, ew, 0.0)
            ew_v[pl.ds(o, L)] = ew
            return d16r

        def _chunk(k, _):
            eb = k * 128
            for j in range(8):
                dstbuf[0, pl.ds(j * L, L)] = _ew16(eb + j * L)
            pltpu.sync_copy(ew_v.at[pl.ds(eb, 128)],
                            deg_sp.at[dstbuf.at[0]], add=True)
            return _
        lax.fori_loop(0, nch, _chunk, None)
        for j in range(tail // L):
            dstbuf_t[0, pl.ds(j * L, L)] = _ew16(nch * 128 + j * L)
        pltpu.sync_copy(ew_v.at[pl.ds(nch * 128, tail)],
                        deg_sp.at[dstbuf_t.at[0]], add=True)

        pltpu.sync_copy(ew_v, ew_h.at[pl.ds(ebase, EPW)])
        plsc.subcore_barrier()

        pltpu.sync_copy(deg_sp.at[pl.ds(rbase, ROWS_PT)], slice_v)
        pltpu.sync_copy(slice_v, degpart_h.at[cid].at[pl.ds(rbase, ROWS_PT)])

    return edge


def _make_agg(feature_split):
    ept = EPT if feature_split else EPW
    sph = 1920 if feature_split else 960
    nsp = ept // sph
    nch = sph // CH
    ngr = nch // 3
    npc = 2560
    width = 128

    @functools.partial(
        pl.kernel,
        out_type=_f32((2, NPAD, width)),
        mesh=_mesh(),
        compiler_params=_SC_PARAMS,
        scratch_types=dict(
            src_v=pltpu.VMEM((sph,), jnp.int32),
            dst_v=pltpu.VMEM((sph,), jnp.int32),
            ew_v=pltpu.VMEM((sph,), jnp.float32),
            dis_v=pltpu.VMEM((NPAD,), jnp.float32),
            tmp_v=pltpu.VMEM((npc,), jnp.float32),
            buf0=pltpu.VMEM((CH, width), jnp.float32),
            buf1=pltpu.VMEM((CH, width), jnp.float32),
            buf2=pltpu.VMEM((CH, width), jnp.float32),
            db0=pltpu.VMEM((1, CH), jnp.int32),
            db1=pltpu.VMEM((1, CH), jnp.int32),
            db2=pltpu.VMEM((1, CH), jnp.int32),
            bc_v=pltpu.VMEM((width,), jnp.float32),
            acc_sp=pltpu.VMEM_SHARED((NPAD, width), jnp.float32),
            sg0=pltpu.SemaphoreType.DMA,
            sg1=pltpu.SemaphoreType.DMA,
            sg2=pltpu.SemaphoreType.DMA,
            ss0=pltpu.SemaphoreType.DMA,
            ss1=pltpu.SemaphoreType.DMA,
            ss2=pltpu.SemaphoreType.DMA,
        ),
    )
    def agg(src_h, dst_h, ew_h, degpart_h, hwh_h, bch_h, out_h,
            src_v, dst_v, ew_v, dis_v, tmp_v, buf0, buf1, buf2,
            db0, db1, db2, bc_v, acc_sp, sg0, sg1, sg2,
            ss0, ss1, ss2):
        cid = lax.axis_index("c")
        sid = lax.axis_index("s")
        if feature_split:
            ebase = sid * ept
            gref = hwh_h.at[cid]
        else:
            ebase = (sid * 2 + cid) * ept
            gref = hwh_h.at[0]
        rbase = sid * ROWS_PT
        bufs = (buf0, buf1, buf2)
        dbs = (db0, db1, db2)
        sgs = (sg0, sg1, sg2)
        sss = (ss0, ss1, ss2)

        pltpu.sync_copy(degpart_h.at[0], dis_v)
        pltpu.sync_copy(bch_h.at[cid], bc_v)

        for piece in range(NPAD // npc):
            pltpu.sync_copy(degpart_h.at[1].at[pl.ds(piece * npc, npc)], tmp_v)

            def _sum_chunk(i, _):
                o = piece * npc + i * L
                d = dis_v[pl.ds(o, L)] + tmp_v[pl.ds(i * L, L)]
                dis_v[pl.ds(o, L)] = _rsqrt16(d)
                return _
            lax.fori_loop(0, npc // L, _sum_chunk, None)

        if feature_split:
            cz = 1.0
        else:
            cz = jnp.where(cid == 0, 1.0, 0.0)
        for b in range(ROWS_PT // CH):
            rows = rbase + b * CH
            pltpu.sync_copy(gref.at[pl.ds(rows, CH)], buf0)
            for g16 in range(CH // L):
                d16 = dis_v[pl.ds(rows + g16 * L, L)]
                rec16 = d16 * d16 * cz

                def _init_row(r, _):
                    rb = _vbcast(rec16, r)
                    row = g16 * L + r
                    for j in range(width // L):
                        buf0[row, pl.ds(j * L, L)] = (
                            buf0[row, pl.ds(j * L, L)] * rb)
                    return _
                lax.fori_loop(0, L, _init_row, None, unroll=4)
            pltpu.sync_copy(buf0, acc_sp.at[pl.ds(rows, CH)])
        plsc.subcore_barrier()

        def _start_g(k, ph):
            pltpu.async_copy(gref.at[src_v.at[pl.ds(k * CH, CH)]],
                             bufs[ph], sgs[ph])

        def _wait_g(ph):
            pltpu.make_async_copy(gref.at[pl.ds(0, CH)], bufs[ph],
                                  sgs[ph]).wait()

        def _start_s(ph):
            pltpu.async_copy(bufs[ph], acc_sp.at[dbs[ph].at[0]],
                             sss[ph], add=True)

        def _wait_s(ph):
            pltpu.make_async_copy(bufs[ph], acc_sp.at[pl.ds(0, CH)],
                                  sss[ph]).wait()

        def _step(k, ph, wait_s_old, start_next):
            if wait_s_old:
                _wait_s((ph + 1) % 3)
            if start_next:
                _start_g(k + 1, (ph + 1) % 3)
            buf = bufs[ph]
            db = dbs[ph]
            norms = []
            for j in range(CH // L):
                o = k * CH + j * L
                s16 = src_v[pl.ds(o, L)]
                d16 = dst_v[pl.ds(o, L)]
                n16 = (plsc.load_gather(dis_v, [s16]) * ew_v[pl.ds(o, L)]
                       * plsc.load_gather(dis_v, [d16]))
                db[0, pl.ds(j * L, L)] = d16
                norms.append(n16)
            _wait_g(ph)
            for j in range(CH // L):
                n16 = norms[j]

                def _scale_row(r, _):
                    nb = _vbcast(n16, r)
                    row = j * L + r
                    for jj in range(width // L):
                        buf[row, pl.ds(jj * L, L)] = (
                            buf[row, pl.ds(jj * L, L)] * nb)
                    return _
                lax.fori_loop(0, L, _scale_row, None, unroll=4)
            _start_s(ph)

        def _subphase(sp, _):
            sbase = ebase + sp * sph
            pltpu.sync_copy(src_h.at[pl.ds(sbase, sph)], src_v)
            pltpu.sync_copy(dst_h.at[pl.ds(sbase, sph)], dst_v)
            pltpu.sync_copy(ew_h.at[pl.ds(sbase, sph)], ew_v)
            _start_g(0, 0)
            _step(0, 0, False, True)
            _step(1, 1, False, True)
            _step(2, 2, True, True)

            def _group(g, _):
                _step(g * 3 + 0, 0, True, True)
                _step(g * 3 + 1, 1, True, True)
                _step(g * 3 + 2, 2, True, True)
                return _
            lax.fori_loop(1, ngr - 1, _group, None)
            _step(nch - 3, 0, True, True)
            _step(nch - 2, 1, True, True)
            _step(nch - 1, 2, True, False)
            _wait_s(1)
            _wait_s(2)
            return _
        lax.fori_loop(0, nsp, _subphase, None)
        plsc.subcore_barrier()

        for b in range(ROWS_PT // CH):
            rows = rbase + b * CH
            pltpu.sync_copy(acc_sp.at[pl.ds(rows, CH)], buf0)
            if feature_split:
                def _out_row(r, _):
                    for j in range(width // L):
                        v = buf0[r, pl.ds(j * L, L)] + bc_v[pl.ds(j * L, L)]
                        v = jnp.maximum(v, 0.0)
                        buf0[r, pl.ds(j * L, L)] = v
                    return _
                lax.fori_loop(0, CH, _out_row, None, unroll=2)
            pltpu.sync_copy(buf0, out_h.at[cid].at[pl.ds(rows, CH)])

    return agg


def _make_final():
    fch = 96
    nch = EPW // fch

    @functools.partial(
        pl.kernel,
        out_type=_f32((EPAD, 64)),
        mesh=_mesh(),
        compiler_params=_SC_PARAMS,
        scratch_types=dict(
            d0_v=pltpu.VMEM((EPW,), jnp.int32),
            d1_v=pltpu.VMEM((EPW,), jnp.int32),
            a0=pltpu.VMEM((fch, 128), jnp.float32),
            a1=pltpu.VMEM((fch, 128), jnp.float32),
            b0=pltpu.VMEM((fch, 128), jnp.float32),
            b1=pltpu.VMEM((fch, 128), jnp.float32),
            q0=pltpu.VMEM((fch, 64), jnp.float32),
            q1=pltpu.VMEM((fch, 64), jnp.float32),
            sa0=pltpu.SemaphoreType.DMA,
            sa1=pltpu.SemaphoreType.DMA,
            sb0=pltpu.SemaphoreType.DMA,
            sb1=pltpu.SemaphoreType.DMA,
            sw0=pltpu.SemaphoreType.DMA,
            sw1=pltpu.SemaphoreType.DMA,
        ),
    )
    def final(p_h, d0_h, d1_h, q_h, d0_v, d1_v, a0, a1, b0, b1, q0, q1,
              sa0, sa1, sb0, sb1, sw0, sw1):
        cid = lax.axis_index("c")
        sid = lax.axis_index("s")
        base = (sid * 2 + cid) * EPW
        abufs = (a0, a1)
        bbufs = (b0, b1)
        qbufs = (q0, q1)
        sas = (sa0, sa1)
        sbs = (sb0, sb1)
        sws = (sw0, sw1)

        pltpu.sync_copy(d0_h.at[pl.ds(base, EPW)], d0_v)
        pltpu.sync_copy(d1_h.at[pl.ds(base, EPW)], d1_v)

        def _start_g(k, ph):
            pltpu.async_copy(p_h.at[d0_v.at[pl.ds(k * fch, fch)]],
                             abufs[ph], sas[ph])
            pltpu.async_copy(p_h.at[d1_v.at[pl.ds(k * fch, fch)]],
                             bbufs[ph], sbs[ph])

        def _wait_g(ph):
            pltpu.make_async_copy(p_h.at[pl.ds(0, fch)], abufs[ph],
                                  sas[ph]).wait()
            pltpu.make_async_copy(p_h.at[pl.ds(0, fch)], bbufs[ph],
                                  sbs[ph]).wait()

        def _wait_w(ph):
            pltpu.make_async_copy(qbufs[ph], q_h.at[pl.ds(0, fch)],
                                  sws[ph]).wait()

        def _step(k, ph, wait_w_prev, start_next):
            if start_next:
                _start_g(k + 1, 1 - ph)
            _wait_g(ph)
            av = abufs[ph]
            bv = bbufs[ph]
            qv = qbufs[ph]

            def _sub_row(r, _):
                for j in range(4):
                    qv[r, pl.ds(j * L, L)] = (av[r, pl.ds(j * L, L)]
                                              - bv[r, pl.ds(j * L, L)])
                return _
            lax.fori_loop(0, fch, _sub_row, None, unroll=4)
            if wait_w_prev:
                _wait_w(1 - ph)
            pltpu.async_copy(qv, q_h.at[pl.ds(base + k * fch, fch)], sws[ph])

        _start_g(0, 0)
        _step(0, 0, False, True)

        def _group(g, _):
            _step(g * 2 - 1, 1, True, True)
            _step(g * 2, 0, True, True)
            return _
        lax.fori_loop(1, nch // 2, _group, None)
        _step(nch - 1, 1, True, False)
        _wait_w(1)

    return final


_edge1 = _make_edge(True)
_edge2 = _make_edge(False)
_agg1 = _make_agg(True)
_agg2 = _make_agg(False)
_final = _make_final()


def kernel(x, coords, edge_index, data_edge_index, Wa, ba, Wg, bg, Waf, baf,
           Wc1, bc1, Wm1, bm1, Wc2, bc2, Wf1, bf1, Wf2, bf2):
    f32 = jnp.float32
    xp = jnp.zeros((NPAD, 512), f32).at[:N].set(x)
    cp = jnp.zeros((NPAD, 4), f32).at[:N].set(coords)
    pad_idx = (N + jnp.arange(EPAD - E, dtype=jnp.int32) % (NPAD - N))
    srcp = jnp.concatenate([edge_index[0], pad_idx])
    dstp = jnp.concatenate([edge_index[1], pad_idx])
    d0p = jnp.concatenate([data_edge_index[0], pad_idx])
    d1p = jnp.concatenate([data_edge_index[1], pad_idx])

    grid = NPAD // RB
    wnode = Wa[0].reshape(2, 512).T
    wgnode = Wg[0].reshape(2, 4).T
    hwh1, ns = pl.pallas_call(
        _tc1_body,
        grid=(grid,),
        in_specs=[
            pl.BlockSpec((RB, 512), lambda i: (i, 0)),
            pl.BlockSpec((RB, 4), lambda i: (i, 0)),
            pl.BlockSpec((512, 256), lambda i: (0, 0)),
            pl.BlockSpec((512, 2), lambda i: (0, 0)),
            pl.BlockSpec((4, 2), lambda i: (0, 0)),
        ],
        out_specs=[
            pl.BlockSpec((2, RB, 128), lambda i: (0, i, 0)),
            pl.BlockSpec((RB, 4), lambda i: (i, 0)),
        ],
        out_shape=[_f32((2, NPAD, 128)), _f32((NPAD, 4))],
    )(xp, cp, Wc1.T, wnode, wgnode)

    z16 = jnp.zeros((L,), f32)
    consts1 = jnp.stack([
        z16 + ba[0], z16 + bg[0], z16 + baf[0],
        z16 + Waf[0, 0], z16 + Waf[0, 1],
        z16, z16, z16])
    ew1, degpart1 = _edge1(ns.reshape(-1), srcp, dstp, consts1)
    h = _agg1(srcp, dstp, ew1, degpart1, hwh1, bc1.reshape(2, 128))

    wm = Wm1[0].reshape(2, 256).T
    hw2, ms = pl.pallas_call(
        _tc2_body,
        grid=(grid,),
        in_specs=[
            pl.BlockSpec((1, RB, 128), lambda i: (0, i, 0)),
            pl.BlockSpec((1, RB, 128), lambda i: (1, i, 0)),
            pl.BlockSpec((128, 128), lambda i: (0, 0)),
            pl.BlockSpec((128, 128), lambda i: (0, 0)),
            pl.BlockSpec((256, 2), lambda i: (0, 0)),
        ],
        out_specs=[
            pl.BlockSpec((RB, 128), lambda i: (i, 0)),
            pl.BlockSpec((RB, 2), lambda i: (i, 0)),
        ],
        out_shape=[_f32((NPAD, 128)), _f32((NPAD, 2))],
    )(h, h, Wc2[:, :128].T, Wc2[:, 128:].T, wm)

    consts2 = jnp.stack([z16 + bm1[0]] + [z16] * 7)
    ew2, degpart2 = _edge2(ms.reshape(-1), srcp, dstp, consts2)
    part = _agg2(srcp, dstp, ew2, degpart2, hw2.reshape(1, NPAD, 128),
                 jnp.zeros((2, 128), f32))

    p = pl.pallas_call(
        _tc3_body,
        grid=(grid,),
        in_specs=[
            pl.BlockSpec((1, RB, 128), lambda i: (0, i, 0)),
            pl.BlockSpec((1, RB, 128), lambda i: (1, i, 0)),
            pl.BlockSpec((1, 128), lambda i: (0, 0)),
            pl.BlockSpec((128, 64), lambda i: (0, 0)),
        ],
        out_specs=pl.BlockSpec((RB, 128), lambda i: (i, 0)),
        out_shape=_f32((NPAD, 128)),
    )(part, part, bc2.reshape(1, 128), Wf1.T)

    q = _final(p, d0p, d1p)

    prob = pl.pallas_call(
        _tc4_body,
        grid=(EPAD // RB,),
        in_specs=[
            pl.BlockSpec((RB, 64), lambda i: (i, 0)),
            pl.BlockSpec((1, 64), lambda i: (0, 0)),
            pl.BlockSpec((64, 1), lambda i: (0, 0)),
            pl.BlockSpec((1, 1), lambda i: (0, 0)),
        ],
        out_specs=pl.BlockSpec((RB, 1), lambda i: (i, 0)),
        out_shape=_f32((EPAD, 1)),
    )(q, bf1.reshape(1, 64), Wf2.reshape(64, 1), bf2.reshape(1, 1))

    return prob[:E]
